# Initial kernel scaffold; baseline (speedup 1.0000x reference)
import jax
import jax.numpy as jnp
from jax.experimental import pallas as pl


def kernel(x, posn_embedding):
    raise NotImplementedError("write your pallas kernel here")


# speedup vs baseline: 2.1343x; 2.1343x over previous
import functools

import jax
import jax.numpy as jnp
from jax import lax
from jax.experimental import pallas as pl
from jax.experimental.pallas import tpu as pltpu
from jax.experimental.pallas import tpu_sc as plsc

_B = 4
_L = 8192
_D = 1024
_N = _B * _L
_NC = 2
_NS = 16
_NW = _NC * _NS
_PER_W = _N // _NW
_CHUNK = 64
_NCHUNK = _PER_W // _CHUNK

_mesh = plsc.VectorSubcoreMesh(core_axis_name="c", subcore_axis_name="s")


@functools.partial(
    pl.kernel,
    mesh=_mesh,
    out_type=jax.ShapeDtypeStruct((_N, _D), jnp.float32),
    scratch_types=[
        pltpu.VMEM((_CHUNK,), jnp.int32),
        pltpu.VMEM((_CHUNK, _D), jnp.float32),
        pltpu.SemaphoreType.DMA,
    ],
)
def _gather_rows(idx_hbm, table_hbm, out_hbm, idx_v, rows_v, sem):
    wid = lax.axis_index("s") * _NC + lax.axis_index("c")
    base = wid * _PER_W

    def body(i, carry):
        off = base + i * _CHUNK
        pltpu.sync_copy(idx_hbm.at[pl.ds(off, _CHUNK)], idx_v)
        pltpu.async_copy(table_hbm.at[idx_v], rows_v, sem).wait()
        pltpu.sync_copy(rows_v, out_hbm.at[pl.ds(off, _CHUNK)])
        return carry

    lax.fori_loop(0, _NCHUNK, body, 0)


def kernel(x, posn_embedding):
    idx = x.reshape(-1).astype(jnp.int32)
    out = _gather_rows(idx, posn_embedding)
    return out.reshape(x.shape + (posn_embedding.shape[-1],))


# speedup vs baseline: 2.3084x; 1.0816x over previous
import functools

import jax
import jax.numpy as jnp
from jax import lax
from jax.experimental import pallas as pl
from jax.experimental.pallas import tpu as pltpu
from jax.experimental.pallas import tpu_sc as plsc

_B = 4
_L = 8192
_D = 1024
_N = _B * _L
_NC = 2
_NS = 16
_NW = _NC * _NS
_PER_W = _N // _NW
_CHUNK = 32
_NCHUNK = _PER_W // _CHUNK

_mesh = plsc.VectorSubcoreMesh(core_axis_name="c", subcore_axis_name="s")


@functools.partial(
    pl.kernel,
    mesh=_mesh,
    out_type=jax.ShapeDtypeStruct((_N, _D), jnp.float32),
    scratch_types=[
        pltpu.VMEM((_PER_W,), jnp.int32),
        pltpu.VMEM((_CHUNK, _D), jnp.float32),
        pltpu.VMEM((_CHUNK, _D), jnp.float32),
        pltpu.SemaphoreType.DMA,
        pltpu.SemaphoreType.DMA,
        pltpu.SemaphoreType.DMA,
        pltpu.SemaphoreType.DMA,
    ],
)
def _gather_rows(idx_hbm, table_hbm, out_hbm, idx_v, rows0, rows1,
                 gsem0, gsem1, osem0, osem1):
    wid = lax.axis_index("s") * _NC + lax.axis_index("c")
    base = wid * _PER_W

    rows = (rows0, rows1)
    gs = (gsem0, gsem1)
    os_ = (osem0, osem1)

    pltpu.sync_copy(idx_hbm.at[pl.ds(base, _PER_W)], idx_v)

    def gather(i, b):
        pltpu.async_copy(
            table_hbm.at[idx_v.at[pl.ds(i * _CHUNK, _CHUNK)]], rows[b], gs[b])

    def wait_gather(b):
        pltpu.make_async_copy(
            table_hbm.at[pl.ds(0, _CHUNK)], rows[b], gs[b]).wait()

    def put_out(i, b):
        pltpu.async_copy(
            rows[b], out_hbm.at[pl.ds(base + i * _CHUNK, _CHUNK)], os_[b])

    def wait_out(b):
        pltpu.make_async_copy(
            rows[b], out_hbm.at[pl.ds(0, _CHUNK)], os_[b]).wait()

    gather(0, 0)
    wait_gather(0)
    gather(1, 1)
    put_out(0, 0)

    def outer(k, carry):
        g = 1 + 2 * k
        for b in range(2):
            i = g + b
            cur = 1 - b
            nxt = b
            wait_gather(cur)
            wait_out(nxt)
            gather(i + 1, nxt)
            put_out(i, cur)
        return carry

    lax.fori_loop(0, (_NCHUNK - 2) // 2, outer, 0)

    wait_gather(1)
    wait_out(0)
    put_out(_NCHUNK - 1, 1)
    wait_out(1)


def kernel(x, posn_embedding):
    idx = x.reshape(-1).astype(jnp.int32)
    out = _gather_rows(idx, posn_embedding)
    return out.reshape(x.shape + (posn_embedding.shape[-1],))


# speedup vs baseline: 2.3568x; 1.0210x over previous
import functools

import jax
import jax.numpy as jnp
from jax import lax
from jax.experimental import pallas as pl
from jax.experimental.pallas import tpu as pltpu
from jax.experimental.pallas import tpu_sc as plsc

_B = 4
_L = 8192
_D = 1024
_N = _B * _L
_NC = 2
_NS = 16
_NW = _NC * _NS
_PER_W = _N // _NW
_CHUNK = 32
_NCHUNK = _PER_W // _CHUNK

_mesh = plsc.VectorSubcoreMesh(core_axis_name="c", subcore_axis_name="s")


@functools.partial(
    pl.kernel,
    mesh=_mesh,
    out_type=jax.ShapeDtypeStruct((_N, _D), jnp.float32),
    scratch_types=[
        pltpu.VMEM((_PER_W,), jnp.int32),
        pltpu.VMEM((_CHUNK, _D), jnp.float32),
        pltpu.VMEM((_CHUNK, _D), jnp.float32),
        pltpu.VMEM((_CHUNK, _D), jnp.float32),
        pltpu.SemaphoreType.DMA,
        pltpu.SemaphoreType.DMA,
        pltpu.SemaphoreType.DMA,
        pltpu.SemaphoreType.DMA,
        pltpu.SemaphoreType.DMA,
        pltpu.SemaphoreType.DMA,
    ],
)
def _gather_rows(idx_hbm, table_hbm, out_hbm, idx_v, rows0, rows1, rows2,
                 gsem0, gsem1, gsem2, osem0, osem1, osem2):
    wid = lax.axis_index("s") * _NC + lax.axis_index("c")
    base = wid * _PER_W

    rows = (rows0, rows1, rows2)
    gs = (gsem0, gsem1, gsem2)
    os_ = (osem0, osem1, osem2)

    pltpu.sync_copy(idx_hbm.at[pl.ds(base, _PER_W)], idx_v)

    def gather(i, b):
        pltpu.async_copy(
            table_hbm.at[idx_v.at[pl.ds(i * _CHUNK, _CHUNK)]], rows[b], gs[b])

    def wait_gather(b):
        pltpu.make_async_copy(
            table_hbm.at[pl.ds(0, _CHUNK)], rows[b], gs[b]).wait()

    def put_out(i, b):
        pltpu.async_copy(
            rows[b], out_hbm.at[pl.ds(base + i * _CHUNK, _CHUNK)], os_[b])

    def wait_out(b):
        pltpu.make_async_copy(
            rows[b], out_hbm.at[pl.ds(0, _CHUNK)], os_[b]).wait()

    gather(0, 0)
    gather(1, 1)

    wait_gather(0)
    put_out(0, 0)
    gather(2, 2)

    def outer(k, carry):
        g = 1 + 3 * k
        for b in range(3):
            i = g + b
            bi = (1 + b) % 3
            bj = b
            wait_gather(bi)
            put_out(i, bi)
            wait_out(bj)
            gather(i + 2, bj)
        return carry

    lax.fori_loop(0, (_NCHUNK - 4) // 3, outer, 0)

    wait_gather(1)
    put_out(_NCHUNK - 4, 1)
    wait_out(0)
    gather(_NCHUNK - 2, 0)
    wait_gather(2)
    put_out(_NCHUNK - 3, 2)
    wait_out(1)
    gather(_NCHUNK - 1, 1)
    wait_gather(0)
    put_out(_NCHUNK - 2, 0)
    wait_gather(1)
    put_out(_NCHUNK - 1, 1)
    wait_out(2)
    wait_out(0)
    wait_out(1)


def kernel(x, posn_embedding):
    idx = x.reshape(-1).astype(jnp.int32)
    out = _gather_rows(idx, posn_embedding)
    return out.reshape(x.shape + (posn_embedding.shape[-1],))
